# baseline (speedup 1.0000x reference)
import jax
import jax.numpy as jnp
from jax import lax
from jax.experimental import pallas as pl
from jax.experimental.pallas import tpu as pltpu
from jax.experimental.pallas import tpu_sc as plsc

NUM_CORES = 2
NUM_SUBCORES = 16
NW = NUM_CORES * NUM_SUBCORES

D = 64
DP = 128
RPW = 512
RPC = 8
CPW = RPW // RPC
NB = 4
W = 26
FPC = RPC * W


def _gather_kernel(lab_hbm, table_hbm, out_hbm,
                   idx_v, rows0, rows1, rows2, rows3,
                   sg0, sg1, sg2, sg3, so0, so1, so2, so3):
    rows = (rows0, rows1, rows2, rows3)
    sg = (sg0, sg1, sg2, sg3)
    so = (so0, so1, so2, so3)
    wid = lax.axis_index("s") * NUM_CORES + lax.axis_index("c")
    r_base = wid * RPW
    f_base = r_base * W

    def start_gather(c, b):
        for k in range(RPC):
            pltpu.async_copy(
                table_hbm.at[idx_v.at[c * RPC + k]],
                rows[b].at[pl.ds(k * W, W)], sg[b])

    def wait_gather(c, b):
        for k in range(RPC):
            pltpu.make_async_copy(
                table_hbm.at[idx_v.at[c * RPC + k]],
                rows[b].at[pl.ds(k * W, W)], sg[b]).wait()

    def start_out(c, b):
        pltpu.async_copy(
            rows[b], out_hbm.at[pl.ds(f_base + c * FPC, FPC)], so[b])

    def wait_out(b):
        pltpu.make_async_copy(
            rows[b], out_hbm.at[pl.ds(f_base, FPC)], so[b]).wait()

    pltpu.sync_copy(lab_hbm.at[pl.ds(r_base, RPW)], idx_v)

    start_gather(0, 0); start_gather(1, 1); start_gather(2, 2)
    wait_gather(0, 0); start_out(0, 0); start_gather(3, 3)
    wait_gather(1, 1); start_out(1, 1); wait_out(0); start_gather(4, 0)
    wait_gather(2, 2); start_out(2, 2); wait_out(1); start_gather(5, 1)
    wait_gather(3, 3); start_out(3, 3); wait_out(2); start_gather(6, 2)

    def body(i, _):
        for b in range(NB):
            c = NB * i + b
            wait_gather(c, b)
            start_out(c, b)
            wait_out((b + 3) % NB)
            start_gather(c + 3, (b + 3) % NB)
        return _

    lax.fori_loop(1, (CPW - 4) // NB, body, None)

    c0 = CPW - 4
    wait_gather(c0, c0 % NB); start_out(c0, c0 % NB)
    wait_out((c0 + 3) % NB); start_gather(c0 + 3, (c0 + 3) % NB)
    for c in (CPW - 3, CPW - 2, CPW - 1):
        wait_gather(c, c % NB); start_out(c, c % NB)
    for b in range(NB):
        wait_out(b)


def kernel(labels, label_embed_weight):
    B0, B1 = labels.shape
    B = B0 * B1
    assert B0 == NW * RPW and B1 == W and CPW % NB == 0

    table_p = jnp.pad(label_embed_weight, ((0, 0), (0, DP - D)))

    run = pl.kernel(
        _gather_kernel,
        out_type=jax.ShapeDtypeStruct((B, DP), jnp.float32),
        mesh=plsc.VectorSubcoreMesh(
            core_axis_name="c", subcore_axis_name="s",
            num_cores=NUM_CORES, num_subcores=NUM_SUBCORES,
        ),
        scratch_types=[
            pltpu.VMEM((RPW, W), jnp.int32),
            pltpu.VMEM((FPC, DP), jnp.float32),
            pltpu.VMEM((FPC, DP), jnp.float32),
            pltpu.VMEM((FPC, DP), jnp.float32),
            pltpu.VMEM((FPC, DP), jnp.float32),
            pltpu.SemaphoreType.DMA,
            pltpu.SemaphoreType.DMA,
            pltpu.SemaphoreType.DMA,
            pltpu.SemaphoreType.DMA,
            pltpu.SemaphoreType.DMA,
            pltpu.SemaphoreType.DMA,
            pltpu.SemaphoreType.DMA,
            pltpu.SemaphoreType.DMA,
        ],
        compiler_params=pltpu.CompilerParams(use_tc_tiling_on_sc=False),
    )
    out_p = run(labels, table_p)
    return out_p[:, :D].reshape(B0, B1, D)


# speedup vs baseline: 1.0061x; 1.0061x over previous
import jax
import jax.numpy as jnp
from jax import lax
from jax.experimental import pallas as pl
from jax.experimental.pallas import tpu as pltpu
from jax.experimental.pallas import tpu_sc as plsc

NUM_CORES = 2
NUM_SUBCORES = 16
NW = NUM_CORES * NUM_SUBCORES

D = 64
DP = 128
RPW = 512
RPC = 4
CPW = RPW // RPC
NB = 4
W = 26
FPC = RPC * W


def _gather_kernel(lab_hbm, table_hbm, out_hbm,
                   idx_v, rows0, rows1, rows2, rows3,
                   sg0, sg1, sg2, sg3, so0, so1, so2, so3):
    rows = (rows0, rows1, rows2, rows3)
    sg = (sg0, sg1, sg2, sg3)
    so = (so0, so1, so2, so3)
    wid = lax.axis_index("s") * NUM_CORES + lax.axis_index("c")
    r_base = wid * RPW
    f_base = r_base * W

    def start_gather(c, b):
        for k in range(RPC):
            pltpu.async_copy(
                table_hbm.at[idx_v.at[c * RPC + k]],
                rows[b].at[pl.ds(k * W, W)], sg[b])

    def wait_gather(c, b):
        for k in range(RPC):
            pltpu.make_async_copy(
                table_hbm.at[idx_v.at[c * RPC + k]],
                rows[b].at[pl.ds(k * W, W)], sg[b]).wait()

    def start_out(c, b):
        pltpu.async_copy(
            rows[b], out_hbm.at[pl.ds(f_base + c * FPC, FPC)], so[b])

    def wait_out(b):
        pltpu.make_async_copy(
            rows[b], out_hbm.at[pl.ds(f_base, FPC)], so[b]).wait()

    pltpu.sync_copy(lab_hbm.at[pl.ds(r_base, RPW)], idx_v)

    start_gather(0, 0); start_gather(1, 1); start_gather(2, 2)
    wait_gather(0, 0); start_out(0, 0); start_gather(3, 3)
    wait_gather(1, 1); start_out(1, 1); wait_out(0); start_gather(4, 0)
    wait_gather(2, 2); start_out(2, 2); wait_out(1); start_gather(5, 1)
    wait_gather(3, 3); start_out(3, 3); wait_out(2); start_gather(6, 2)

    def body(i, _):
        for b in range(NB):
            c = NB * i + b
            wait_gather(c, b)
            start_out(c, b)
            wait_out((b + 3) % NB)
            start_gather(c + 3, (b + 3) % NB)
        return _

    lax.fori_loop(1, (CPW - 4) // NB, body, None)

    c0 = CPW - 4
    wait_gather(c0, c0 % NB); start_out(c0, c0 % NB)
    wait_out((c0 + 3) % NB); start_gather(c0 + 3, (c0 + 3) % NB)
    for c in (CPW - 3, CPW - 2, CPW - 1):
        wait_gather(c, c % NB); start_out(c, c % NB)
    for b in range(NB):
        wait_out(b)


def kernel(labels, label_embed_weight):
    B0, B1 = labels.shape
    B = B0 * B1
    assert B0 == NW * RPW and B1 == W and CPW % NB == 0

    table_p = jnp.pad(label_embed_weight, ((0, 0), (0, DP - D)))

    run = pl.kernel(
        _gather_kernel,
        out_type=jax.ShapeDtypeStruct((B, DP), jnp.float32),
        mesh=plsc.VectorSubcoreMesh(
            core_axis_name="c", subcore_axis_name="s",
            num_cores=NUM_CORES, num_subcores=NUM_SUBCORES,
        ),
        scratch_types=[
            pltpu.VMEM((RPW, W), jnp.int32),
            pltpu.VMEM((FPC, DP), jnp.float32),
            pltpu.VMEM((FPC, DP), jnp.float32),
            pltpu.VMEM((FPC, DP), jnp.float32),
            pltpu.VMEM((FPC, DP), jnp.float32),
            pltpu.SemaphoreType.DMA,
            pltpu.SemaphoreType.DMA,
            pltpu.SemaphoreType.DMA,
            pltpu.SemaphoreType.DMA,
            pltpu.SemaphoreType.DMA,
            pltpu.SemaphoreType.DMA,
            pltpu.SemaphoreType.DMA,
            pltpu.SemaphoreType.DMA,
        ],
        compiler_params=pltpu.CompilerParams(use_tc_tiling_on_sc=True),
    )
    out_p = run(labels, table_p)
    return out_p[:, :D].reshape(B0, B1, D)


# speedup vs baseline: 1.0645x; 1.0581x over previous
import jax
import jax.numpy as jnp
from jax import lax
from jax.experimental import pallas as pl
from jax.experimental.pallas import tpu as pltpu
from jax.experimental.pallas import tpu_sc as plsc

NUM_CORES = 2
NUM_SUBCORES = 16
NW = NUM_CORES * NUM_SUBCORES

D = 64
RPW = 512
RPC = 16
CPW = RPW // RPC
NB = 3


def _gather_kernel(lab_hbm, table_hbm, out_hbm,
                   idx_v, rows0, rows1, rows2,
                   sg0, sg1, sg2, so0, so1, so2):
    W = lab_hbm.shape[1]
    rows = (rows0, rows1, rows2)
    sg = (sg0, sg1, sg2)
    so = (so0, so1, so2)
    wid = lax.axis_index("s") * NUM_CORES + lax.axis_index("c")
    r_base = wid * RPW

    def start_gather(c, b):
        for k in range(RPC):
            pltpu.async_copy(
                table_hbm.at[idx_v.at[c * RPC + k]], rows[b].at[k], sg[b])

    def wait_gather(c, b):
        for k in range(RPC):
            pltpu.make_async_copy(
                table_hbm.at[idx_v.at[c * RPC + k]], rows[b].at[k], sg[b]
            ).wait()

    def start_out(c, b):
        pltpu.async_copy(
            rows[b], out_hbm.at[pl.ds(r_base + c * RPC, RPC)], so[b])

    def wait_out(b):
        pltpu.make_async_copy(
            rows[b], out_hbm.at[pl.ds(r_base, RPC)], so[b]).wait()

    pltpu.sync_copy(lab_hbm.at[pl.ds(r_base, RPW)], idx_v)

    start_gather(0, 0)
    start_gather(1, 1)
    wait_gather(0, 0); start_out(0, 0); start_gather(2, 2)
    wait_gather(1, 1); start_out(1, 1); wait_out(0); start_gather(3, 0)
    wait_gather(2, 2); start_out(2, 2); wait_out(1); start_gather(4, 1)

    def body(i, _):
        for b in range(NB):
            c = NB * i + b
            wait_gather(c, b)
            start_out(c, b)
            wait_out((b + 2) % NB)
            start_gather(c + 2, (b + 2) % NB)
        return _

    lax.fori_loop(1, (CPW - 2) // NB, body, None)

    c0, c1 = CPW - 2, CPW - 1
    b0, b1 = c0 % NB, c1 % NB
    wait_gather(c0, b0); start_out(c0, b0)
    wait_gather(c1, b1); start_out(c1, b1)
    wait_out((b1 + 1) % NB); wait_out(b0); wait_out(b1)


def kernel(labels, label_embed_weight):
    B0, B1 = labels.shape
    assert B0 == NW * RPW and CPW % NB == 2 % NB

    run = pl.kernel(
        _gather_kernel,
        out_type=jax.ShapeDtypeStruct((B0, B1, D), jnp.float32),
        mesh=plsc.VectorSubcoreMesh(
            core_axis_name="c", subcore_axis_name="s",
            num_cores=NUM_CORES, num_subcores=NUM_SUBCORES,
        ),
        scratch_types=[
            pltpu.VMEM((RPW, B1), jnp.int32),
            pltpu.VMEM((RPC, B1, D), jnp.float32),
            pltpu.VMEM((RPC, B1, D), jnp.float32),
            pltpu.VMEM((RPC, B1, D), jnp.float32),
            pltpu.SemaphoreType.DMA,
            pltpu.SemaphoreType.DMA,
            pltpu.SemaphoreType.DMA,
            pltpu.SemaphoreType.DMA,
            pltpu.SemaphoreType.DMA,
            pltpu.SemaphoreType.DMA,
        ],
        compiler_params=pltpu.CompilerParams(use_tc_tiling_on_sc=False),
    )
    return run(labels, label_embed_weight)
